# baseline (speedup 1.0000x reference)
import jax
import jax.numpy as jnp
from jax.experimental import pallas as pl

_NB = 8


def _bf16_dot(a, b, dims):
    return jax.lax.dot_general(
        a.astype(jnp.bfloat16), b.astype(jnp.bfloat16), (dims, ((), ())),
        preferred_element_type=jnp.float32)


def _vt_kernel(x_ref, anchor_ref, ls_ref, box_ref, out_ref):
    B = x_ref.shape[0]
    K = anchor_ref.shape[1]
    f32 = jnp.float32

    for j in range(_NB):
        ar = anchor_ref[j]
        bc = box_ref[j]
        box_max = jax.nn.softplus(bc[:, 0]) + 1.0
        box_min = -(jax.nn.softplus(bc[:, 1]) + 1.0)
        pts = ar / (1.0 + jnp.abs(ar))
        pts = (pts + 1.0) / 2.0
        pts = pts * (box_max - box_min)[None, :] + box_min[None, :]
        p2 = jnp.sum(pts * pts, axis=1)

        xb = x_ref[:, j, :]
        x2 = jnp.sum(xb * xb, axis=1, keepdims=True)
        s = _bf16_dot(xb, pts, ((1,), (1,)))
        d2 = x2 - 2.0 * s + p2[None, :]

        iota_k = jax.lax.broadcasted_iota(jnp.int32, (B, K), 1)
        dmin = jnp.min(d2, axis=1, keepdims=True)
        nearest = jnp.min(jnp.where(d2 == dmin, iota_k, K), axis=1, keepdims=True)
        onehot = (iota_k == nearest).astype(f32)

        pts_bf = pts.astype(jnp.bfloat16).astype(f32)
        x_k = _bf16_dot(onehot, pts_bf, ((1,), (0,)))
        diff = xb - x_k
        dist = jnp.sqrt(jnp.sum(diff * diff, axis=1, keepdims=True))
        del_x = diff / (dist + 1e-6)

        dv = jnp.concatenate([del_x, x_k], axis=0)
        uv = _bf16_dot(pts, dv, ((1,), (1,)))
        u = uv[:, :B].T
        v = uv[:, B:].T
        xk_dx = jnp.sum(x_k * del_x, axis=1, keepdims=True)
        xk2 = jnp.sum(x_k * x_k, axis=1, keepdims=True)
        g_vor = 2.0 * (u - xk_dx)
        h_vor = p2[None, :] - 2.0 * v + xk2
        l_vor = h_vor / g_vor
        l_vor = jnp.where(l_vor > 0, l_vor, jnp.inf)
        lamb = jnp.min(l_vor, axis=1, keepdims=True)

        h_hi = box_max[None, :] - x_k
        h_lo = x_k - box_min[None, :]
        l_hi = h_hi / del_x
        l_lo = h_lo / (-del_x)
        l_hi = jnp.where(l_hi > 0, l_hi, jnp.inf)
        l_lo = jnp.where(l_lo > 0, l_lo, jnp.inf)
        lamb = jnp.minimum(lamb, jnp.min(l_hi, axis=1, keepdims=True))
        lamb = jnp.minimum(lamb, jnp.min(l_lo, axis=1, keepdims=True))

        exp_ls = jnp.exp(ls_ref[j])
        scale = jnp.sum(onehot * exp_ls[None, :], axis=1, keepdims=True)
        t = dist * scale
        alpha = t / (1.0 + t)
        x_lamb = x_k + lamb * del_x
        out_ref[:, j, :] = x_k + alpha * (x_lamb - x_k)


@jax.jit
def kernel(x, anchor_raw, log_scale, box_constraints):
    B, N, D = x.shape
    K = anchor_raw.shape[1]
    grid = (N // _NB,)
    return pl.pallas_call(
        _vt_kernel,
        grid=grid,
        in_specs=[
            pl.BlockSpec((B, _NB, D), lambda i: (0, i, 0)),
            pl.BlockSpec((_NB, K, D), lambda i: (i, 0, 0)),
            pl.BlockSpec((_NB, K), lambda i: (i, 0)),
            pl.BlockSpec((_NB, D, 2), lambda i: (i, 0, 0)),
        ],
        out_specs=pl.BlockSpec((B, _NB, D), lambda i: (0, i, 0)),
        out_shape=jax.ShapeDtypeStruct((B, N, D), jnp.float32),
    )(x, anchor_raw, log_scale, box_constraints)


# speedup vs baseline: 1.2405x; 1.2405x over previous
import jax
import jax.numpy as jnp
from jax.experimental import pallas as pl

_NB = 4


def _bf16_dot(a, b):
    return jax.lax.dot_general(
        a.astype(jnp.bfloat16), b.astype(jnp.bfloat16),
        (((1,), (1,)), ((), ())), preferred_element_type=jnp.float32)


def _vt_kernel(x_ref, anchor_ref, ls_ref, box_ref, out_ref):
    NB, B, D = x_ref.shape
    K = anchor_ref.shape[1]
    R = NB * B
    f32 = jnp.float32

    ar = anchor_ref[...]
    bc = box_ref[...]
    box_max = jax.nn.softplus(bc[:, :, 0]) + 1.0
    box_min = -(jax.nn.softplus(bc[:, :, 1]) + 1.0)
    pts = ar / (1.0 + jnp.abs(ar))
    pts = (pts + 1.0) / 2.0
    pts = pts * (box_max[:, None, :] - box_min[:, None, :]) + box_min[:, None, :]
    p2 = jnp.sum(pts * pts, axis=2)
    pts2d = pts.reshape(NB * K, D)
    pts_bf = pts2d.astype(jnp.bfloat16).astype(f32)

    xs = x_ref[...].reshape(R, D)
    x2 = jnp.sum(xs * xs, axis=1, keepdims=True)

    s_all = _bf16_dot(xs, pts2d)
    s = jnp.concatenate(
        [s_all[j * B:(j + 1) * B, j * K:(j + 1) * K] for j in range(NB)],
        axis=0)

    p2_rows = jnp.broadcast_to(p2[:, None, :], (NB, B, K)).reshape(R, K)
    exp_ls_rows = jnp.broadcast_to(
        jnp.exp(ls_ref[:, 0, :])[:, None, :], (NB, B, K)).reshape(R, K)
    bmax_rows = jnp.broadcast_to(box_max[:, None, :], (NB, B, D)).reshape(R, D)
    bmin_rows = jnp.broadcast_to(box_min[:, None, :], (NB, B, D)).reshape(R, D)

    d2 = x2 - 2.0 * s + p2_rows
    iota_k = jax.lax.broadcasted_iota(jnp.int32, (R, K), 1)
    dmin = jnp.min(d2, axis=1, keepdims=True)
    nearest = jnp.min(jnp.where(d2 == dmin, iota_k, K), axis=1, keepdims=True)
    onehot = (iota_k == nearest).astype(f32)

    rown = jax.lax.broadcasted_iota(jnp.int32, (R, 1), 0) // B
    oh_big = jnp.concatenate(
        [jnp.where(rown == j, onehot, 0.0) for j in range(NB)], axis=1)
    x_k = _bf16_dot(oh_big, pts_bf.T)

    diff = xs - x_k
    dist = jnp.sqrt(jnp.sum(diff * diff, axis=1, keepdims=True))
    del_x = diff / (dist + 1e-6)

    dv = jnp.concatenate([del_x, x_k], axis=0)
    uv_all = _bf16_dot(dv, pts2d)
    u = jnp.concatenate(
        [uv_all[j * B:(j + 1) * B, j * K:(j + 1) * K] for j in range(NB)],
        axis=0)
    v = jnp.concatenate(
        [uv_all[R + j * B:R + (j + 1) * B, j * K:(j + 1) * K] for j in range(NB)],
        axis=0)

    xk_dx = jnp.sum(x_k * del_x, axis=1, keepdims=True)
    xk2 = jnp.sum(x_k * x_k, axis=1, keepdims=True)
    g_vor = 2.0 * (u - xk_dx)
    h_vor = p2_rows - 2.0 * v + xk2
    l_vor = h_vor / g_vor
    l_vor = jnp.where(l_vor > 0, l_vor, jnp.inf)
    lamb = jnp.min(l_vor, axis=1, keepdims=True)

    l_hi = (bmax_rows - x_k) / del_x
    l_lo = (x_k - bmin_rows) / (-del_x)
    l_hi = jnp.where(l_hi > 0, l_hi, jnp.inf)
    l_lo = jnp.where(l_lo > 0, l_lo, jnp.inf)
    lamb = jnp.minimum(lamb, jnp.min(l_hi, axis=1, keepdims=True))
    lamb = jnp.minimum(lamb, jnp.min(l_lo, axis=1, keepdims=True))

    scale = jnp.sum(onehot * exp_ls_rows, axis=1, keepdims=True)
    t = dist * scale
    alpha = t / (1.0 + t)
    x_lamb = x_k + lamb * del_x
    z = x_k + alpha * (x_lamb - x_k)

    out_ref[...] = z.reshape(NB, B, D)


@jax.jit
def kernel(x, anchor_raw, log_scale, box_constraints):
    B, N, D = x.shape
    K = anchor_raw.shape[1]
    xt = jnp.transpose(x, (1, 0, 2))
    ls3 = log_scale.reshape(N, 1, K)
    grid = (N // _NB,)
    zt = pl.pallas_call(
        _vt_kernel,
        grid=grid,
        in_specs=[
            pl.BlockSpec((_NB, B, D), lambda i: (i, 0, 0)),
            pl.BlockSpec((_NB, K, D), lambda i: (i, 0, 0)),
            pl.BlockSpec((_NB, 1, K), lambda i: (i, 0, 0)),
            pl.BlockSpec((_NB, D, 2), lambda i: (i, 0, 0)),
        ],
        out_specs=pl.BlockSpec((_NB, B, D), lambda i: (i, 0, 0)),
        out_shape=jax.ShapeDtypeStruct((N, B, D), jnp.float32),
    )(xt, anchor_raw, ls3, box_constraints)
    return jnp.transpose(zt, (1, 0, 2))


# speedup vs baseline: 4.9572x; 3.9963x over previous
import jax
import jax.numpy as jnp
from jax.experimental import pallas as pl

_NB = 8


def _bf16_dot(a, b, dims):
    return jax.lax.dot_general(
        a.astype(jnp.bfloat16), b.astype(jnp.bfloat16), (dims, ((), ())),
        preferred_element_type=jnp.float32)


def _fast_div(h, g):
    r = pl.reciprocal(g, approx=True)
    r = r * (2.0 - g * r)
    r = r * (2.0 - g * r)
    return h * r


def _vt_kernel(x_ref, anchor_ref, ls_ref, box_ref, out_ref):
    NB, B, D = x_ref.shape
    K = anchor_ref.shape[1]
    f32 = jnp.float32
    ones_row = jnp.ones((1, D), dtype=f32)

    for j in range(NB):
        ar = anchor_ref[j]
        box_max = jax.nn.softplus(box_ref[j, 0:1, :]) + 1.0
        box_min = -(jax.nn.softplus(box_ref[j, 1:2, :]) + 1.0)
        pts = ar / (1.0 + jnp.abs(ar))
        pts = (pts + 1.0) / 2.0
        pts = pts * (box_max - box_min) + box_min
        p2_row = jax.lax.dot_general(
            ones_row, pts * pts, (((1,), (1,)), ((), ())),
            precision=jax.lax.Precision.HIGHEST,
            preferred_element_type=f32)

        xb = x_ref[j]
        x2 = jnp.sum(xb * xb, axis=1, keepdims=True)
        s = _bf16_dot(xb, pts, ((1,), (1,)))
        d2 = x2 - 2.0 * s + p2_row

        iota_k = jax.lax.broadcasted_iota(jnp.int32, (B, K), 1)
        dmin = jnp.min(d2, axis=1, keepdims=True)
        nearest = jnp.min(jnp.where(d2 == dmin, iota_k, K), axis=1, keepdims=True)
        onehot = (iota_k == nearest).astype(f32)

        pts_bf = pts.astype(jnp.bfloat16).astype(f32)
        x_k = _bf16_dot(onehot, pts_bf, ((1,), (0,)))
        diff = xb - x_k
        dist = jnp.sqrt(jnp.sum(diff * diff, axis=1, keepdims=True))
        del_x = _fast_div(diff, dist + 1e-6)

        dv = jnp.concatenate([del_x, x_k], axis=0)
        uv = _bf16_dot(dv, pts, ((1,), (1,)))
        u = uv[:B]
        v = uv[B:]
        xk_dx = jnp.sum(x_k * del_x, axis=1, keepdims=True)
        xk2 = jnp.sum(x_k * x_k, axis=1, keepdims=True)
        g_vor = 2.0 * (u - xk_dx)
        h_vor = p2_row - 2.0 * v + xk2
        l_vor = _fast_div(h_vor, g_vor)
        l_vor = jnp.where(l_vor > 0, l_vor, jnp.inf)
        lamb = jnp.min(l_vor, axis=1, keepdims=True)

        r_del = _fast_div(jnp.float32(1.0), del_x)
        l_hi = (box_max - x_k) * r_del
        l_lo = (x_k - box_min) * (-r_del)
        l_hi = jnp.where(l_hi > 0, l_hi, jnp.inf)
        l_lo = jnp.where(l_lo > 0, l_lo, jnp.inf)
        lamb = jnp.minimum(lamb, jnp.min(l_hi, axis=1, keepdims=True))
        lamb = jnp.minimum(lamb, jnp.min(l_lo, axis=1, keepdims=True))

        ls_sel = jnp.sum(onehot * ls_ref[j], axis=1, keepdims=True)
        scale = jnp.exp(ls_sel)
        t = dist * scale
        alpha = t / (1.0 + t)
        x_lamb = x_k + lamb * del_x
        out_ref[j] = x_k + alpha * (x_lamb - x_k)


@jax.jit
def kernel(x, anchor_raw, log_scale, box_constraints):
    B, N, D = x.shape
    K = anchor_raw.shape[1]
    xt = jnp.transpose(x, (1, 0, 2))
    ls3 = log_scale.reshape(N, 1, K)
    box_t = jnp.transpose(box_constraints, (0, 2, 1))
    grid = (N // _NB,)
    zt = pl.pallas_call(
        _vt_kernel,
        grid=grid,
        in_specs=[
            pl.BlockSpec((_NB, B, D), lambda i: (i, 0, 0)),
            pl.BlockSpec((_NB, K, D), lambda i: (i, 0, 0)),
            pl.BlockSpec((_NB, 1, K), lambda i: (i, 0, 0)),
            pl.BlockSpec((_NB, 2, D), lambda i: (i, 0, 0)),
        ],
        out_specs=pl.BlockSpec((_NB, B, D), lambda i: (i, 0, 0)),
        out_shape=jax.ShapeDtypeStruct((N, B, D), jnp.float32),
    )(xt, anchor_raw, ls3, box_t)
    return jnp.transpose(zt, (1, 0, 2))


# speedup vs baseline: 9.2648x; 1.8690x over previous
import jax
import jax.numpy as jnp
from jax.experimental import pallas as pl

_NB = 8


def _bf16_dot(a, b, dims):
    return jax.lax.dot_general(
        a.astype(jnp.bfloat16), b.astype(jnp.bfloat16), (dims, ((), ())),
        preferred_element_type=jnp.float32)


def _fast_div(h, g):
    r = pl.reciprocal(g, approx=True)
    r = r * (2.0 - g * r)
    r = r * (2.0 - g * r)
    return h * r


def _vt_kernel(x_ref, anchor_ref, ls_ref, box_ref, out_ref):
    NB, B, D = x_ref.shape
    K = anchor_ref.shape[1]
    f32 = jnp.float32
    ones_row = jnp.ones((1, D), dtype=f32)
    iota_k = jax.lax.broadcasted_iota(jnp.int32, (B, K), 1)
    J = range(NB)

    box_max = [jax.nn.softplus(box_ref[j, 0:1, :]) + 1.0 for j in J]
    box_min = [-(jax.nn.softplus(box_ref[j, 1:2, :]) + 1.0) for j in J]
    pts = [anchor_ref[j] for j in J]
    pts = [p / (1.0 + jnp.abs(p)) for p in pts]
    pts = [(p + 1.0) / 2.0 for p in pts]
    pts = [p * (bx - bn) + bn for p, bx, bn in zip(pts, box_max, box_min)]
    p2_row = [jax.lax.dot_general(
        ones_row, p * p, (((1,), (1,)), ((), ())),
        precision=jax.lax.Precision.HIGHEST,
        preferred_element_type=f32) for p in pts]

    xb = [x_ref[j] for j in J]
    x2 = [jnp.sum(x * x, axis=1, keepdims=True) for x in xb]
    s = [_bf16_dot(x, p, ((1,), (1,))) for x, p in zip(xb, pts)]
    d2 = [a - 2.0 * b + c for a, b, c in zip(x2, s, p2_row)]

    dmin = [jnp.min(d, axis=1, keepdims=True) for d in d2]
    nearest = [jnp.min(jnp.where(d == m, iota_k, K), axis=1, keepdims=True)
               for d, m in zip(d2, dmin)]
    onehot = [(iota_k == nr).astype(f32) for nr in nearest]

    pts_bf = [p.astype(jnp.bfloat16).astype(f32) for p in pts]
    x_k = [_bf16_dot(oh, pb, ((1,), (0,)))
           for oh, pb in zip(onehot, pts_bf)]
    diff = [x - k for x, k in zip(xb, x_k)]
    dist = [jnp.sqrt(jnp.sum(df * df, axis=1, keepdims=True)) for df in diff]
    del_x = [_fast_div(df, ds + 1e-6) for df, ds in zip(diff, dist)]

    dv = [jnp.concatenate([dx, k], axis=0) for dx, k in zip(del_x, x_k)]
    uv = [_bf16_dot(d, p, ((1,), (1,))) for d, p in zip(dv, pts)]
    xk_dx = [jnp.sum(k * dx, axis=1, keepdims=True)
             for k, dx in zip(x_k, del_x)]
    xk2 = [jnp.sum(k * k, axis=1, keepdims=True) for k in x_k]
    g_vor = [2.0 * (w[:B] - a) for w, a in zip(uv, xk_dx)]
    h_vor = [c - 2.0 * w[B:] + b for c, w, b in zip(p2_row, uv, xk2)]
    l_vor = [_fast_div(h, g) for h, g in zip(h_vor, g_vor)]
    l_vor = [jnp.where(l > 0, l, jnp.inf) for l in l_vor]
    lamb = [jnp.min(l, axis=1, keepdims=True) for l in l_vor]

    r_del = [_fast_div(jnp.float32(1.0), dx) for dx in del_x]
    l_hi = [(bx - k) * r for bx, k, r in zip(box_max, x_k, r_del)]
    l_lo = [(k - bn) * (-r) for k, bn, r in zip(x_k, box_min, r_del)]
    l_hi = [jnp.where(l > 0, l, jnp.inf) for l in l_hi]
    l_lo = [jnp.where(l > 0, l, jnp.inf) for l in l_lo]
    lamb = [jnp.minimum(a, jnp.min(l, axis=1, keepdims=True))
            for a, l in zip(lamb, l_hi)]
    lamb = [jnp.minimum(a, jnp.min(l, axis=1, keepdims=True))
            for a, l in zip(lamb, l_lo)]

    ls_sel = [jnp.sum(oh * ls_ref[j], axis=1, keepdims=True)
              for j, oh in zip(J, onehot)]
    scale = [jnp.exp(v) for v in ls_sel]
    t = [ds * sc for ds, sc in zip(dist, scale)]
    alpha = [a / (1.0 + a) for a in t]
    x_lamb = [k + lm * dx for k, lm, dx in zip(x_k, lamb, del_x)]
    for j in J:
        out_ref[j] = x_k[j] + alpha[j] * (x_lamb[j] - x_k[j])


@jax.jit
def kernel(x, anchor_raw, log_scale, box_constraints):
    B, N, D = x.shape
    K = anchor_raw.shape[1]
    xt = jnp.transpose(x, (1, 0, 2))
    ls3 = log_scale.reshape(N, 1, K)
    box_t = jnp.transpose(box_constraints, (0, 2, 1))
    grid = (N // _NB,)
    zt = pl.pallas_call(
        _vt_kernel,
        grid=grid,
        in_specs=[
            pl.BlockSpec((_NB, B, D), lambda i: (i, 0, 0)),
            pl.BlockSpec((_NB, K, D), lambda i: (i, 0, 0)),
            pl.BlockSpec((_NB, 1, K), lambda i: (i, 0, 0)),
            pl.BlockSpec((_NB, 2, D), lambda i: (i, 0, 0)),
        ],
        out_specs=pl.BlockSpec((_NB, B, D), lambda i: (i, 0, 0)),
        out_shape=jax.ShapeDtypeStruct((N, B, D), jnp.float32),
    )(xt, anchor_raw, ls3, box_t)
    return jnp.transpose(zt, (1, 0, 2))


# speedup vs baseline: 9.9623x; 1.0753x over previous
import jax
import jax.numpy as jnp
from jax.experimental import pallas as pl

_NB = 16


def _bf16_dot(a, b, dims):
    return jax.lax.dot_general(
        a.astype(jnp.bfloat16), b.astype(jnp.bfloat16), (dims, ((), ())),
        preferred_element_type=jnp.float32)


def _fast_div(h, g):
    r = pl.reciprocal(g, approx=True)
    r = r * (2.0 - g * r)
    r = r * (2.0 - g * r)
    return h * r


def _vt_kernel(x_ref, anchor_ref, ls_ref, box_ref, out_ref):
    NB, B, D = x_ref.shape
    K = anchor_ref.shape[1]
    f32 = jnp.float32
    ones_row = jnp.ones((1, D), dtype=f32)
    iota_k = jax.lax.broadcasted_iota(jnp.int32, (B, K), 1)
    J = range(NB)

    box_max = [jax.nn.softplus(box_ref[j, 0:1, :]) + 1.0 for j in J]
    box_min = [-(jax.nn.softplus(box_ref[j, 1:2, :]) + 1.0) for j in J]
    pts = [anchor_ref[j] for j in J]
    pts = [p / (1.0 + jnp.abs(p)) for p in pts]
    pts = [(p + 1.0) / 2.0 for p in pts]
    pts = [p * (bx - bn) + bn for p, bx, bn in zip(pts, box_max, box_min)]
    p2_row = [jax.lax.dot_general(
        ones_row, p * p, (((1,), (1,)), ((), ())),
        precision=jax.lax.Precision.HIGHEST,
        preferred_element_type=f32) for p in pts]

    xb = [x_ref[j] for j in J]
    x2 = [jnp.sum(x * x, axis=1, keepdims=True) for x in xb]
    s = [_bf16_dot(x, p, ((1,), (1,))) for x, p in zip(xb, pts)]
    d2 = [a - 2.0 * b + c for a, b, c in zip(x2, s, p2_row)]

    dmin = [jnp.min(d, axis=1, keepdims=True) for d in d2]
    nearest = [jnp.min(jnp.where(d == m, iota_k, K), axis=1, keepdims=True)
               for d, m in zip(d2, dmin)]
    onehot = [(iota_k == nr).astype(f32) for nr in nearest]

    pts_bf = [p.astype(jnp.bfloat16).astype(f32) for p in pts]
    x_k = [_bf16_dot(oh, pb, ((1,), (0,)))
           for oh, pb in zip(onehot, pts_bf)]
    diff = [x - k for x, k in zip(xb, x_k)]
    dist = [jnp.sqrt(jnp.sum(df * df, axis=1, keepdims=True)) for df in diff]
    del_x = [_fast_div(df, ds + 1e-6) for df, ds in zip(diff, dist)]

    dv = [jnp.concatenate([dx, k], axis=0) for dx, k in zip(del_x, x_k)]
    uv = [_bf16_dot(d, p, ((1,), (1,))) for d, p in zip(dv, pts)]
    xk_dx = [jnp.sum(k * dx, axis=1, keepdims=True)
             for k, dx in zip(x_k, del_x)]
    xk2 = [jnp.sum(k * k, axis=1, keepdims=True) for k in x_k]
    g_vor = [2.0 * (w[:B] - a) for w, a in zip(uv, xk_dx)]
    h_vor = [c - 2.0 * w[B:] + b for c, w, b in zip(p2_row, uv, xk2)]
    l_vor = [_fast_div(h, g) for h, g in zip(h_vor, g_vor)]
    l_vor = [jnp.where(l > 0, l, jnp.inf) for l in l_vor]
    lamb = [jnp.min(l, axis=1, keepdims=True) for l in l_vor]

    r_del = [_fast_div(jnp.float32(1.0), dx) for dx in del_x]
    l_hi = [(bx - k) * r for bx, k, r in zip(box_max, x_k, r_del)]
    l_lo = [(k - bn) * (-r) for k, bn, r in zip(x_k, box_min, r_del)]
    l_hi = [jnp.where(l > 0, l, jnp.inf) for l in l_hi]
    l_lo = [jnp.where(l > 0, l, jnp.inf) for l in l_lo]
    lamb = [jnp.minimum(a, jnp.min(l, axis=1, keepdims=True))
            for a, l in zip(lamb, l_hi)]
    lamb = [jnp.minimum(a, jnp.min(l, axis=1, keepdims=True))
            for a, l in zip(lamb, l_lo)]

    ls_sel = [jnp.sum(oh * ls_ref[j], axis=1, keepdims=True)
              for j, oh in zip(J, onehot)]
    scale = [jnp.exp(v) for v in ls_sel]
    t = [ds * sc for ds, sc in zip(dist, scale)]
    alpha = [a / (1.0 + a) for a in t]
    x_lamb = [k + lm * dx for k, lm, dx in zip(x_k, lamb, del_x)]
    for j in J:
        out_ref[j] = x_k[j] + alpha[j] * (x_lamb[j] - x_k[j])


@jax.jit
def kernel(x, anchor_raw, log_scale, box_constraints):
    B, N, D = x.shape
    K = anchor_raw.shape[1]
    xt = jnp.transpose(x, (1, 0, 2))
    ls3 = log_scale.reshape(N, 1, K)
    box_t = jnp.transpose(box_constraints, (0, 2, 1))
    grid = (N // _NB,)
    zt = pl.pallas_call(
        _vt_kernel,
        grid=grid,
        in_specs=[
            pl.BlockSpec((_NB, B, D), lambda i: (i, 0, 0)),
            pl.BlockSpec((_NB, K, D), lambda i: (i, 0, 0)),
            pl.BlockSpec((_NB, 1, K), lambda i: (i, 0, 0)),
            pl.BlockSpec((_NB, 2, D), lambda i: (i, 0, 0)),
        ],
        out_specs=pl.BlockSpec((_NB, B, D), lambda i: (i, 0, 0)),
        out_shape=jax.ShapeDtypeStruct((N, B, D), jnp.float32),
    )(xt, anchor_raw, ls3, box_t)
    return jnp.transpose(zt, (1, 0, 2))
